# baseline (speedup 1.0000x reference)
import functools

import jax
import jax.numpy as jnp
from jax import lax
from jax.experimental import pallas as pl
from jax.experimental.pallas import tpu as pltpu
from jax.experimental.pallas import tpu_sc as plsc

VOCAB = 1000
VPAD = 1024
EMB = 128
BATCH = 16384

L = 16
RB = 8
NBLK = VOCAB // RB
NGRP = 8
KMAX = 16
HALF = BATCH // 2
CCH = 2048
NCH = HALF // CCH
NIC = CCH // L


def _decode_kernel(lw0_ref, w1_ref, lb0_ref, lw1_ref, w0_ref, lb1_ref,
                   pt0_ref, pt1_ref):
    pt0_ref[:, :VOCAB] = (
        jax.lax.dot_general(lw0_ref[...], w1_ref[...],
                            (((1,), (1,)), ((), ())),
                            preferred_element_type=jnp.float32)
        + lb0_ref[...]
    )
    pt1_ref[:, :VOCAB] = (
        jax.lax.dot_general(lw1_ref[...], w0_ref[...],
                            (((1,), (1,)), ((), ())),
                            preferred_element_type=jnp.float32)
        + lb1_ref[...]
    )


def _decode_tables(W0, W1, Lw0, Lb0, Lw1, Lb1):
    return pl.pallas_call(
        _decode_kernel,
        out_shape=(
            jax.ShapeDtypeStruct((VOCAB, VPAD), jnp.float32),
            jax.ShapeDtypeStruct((VOCAB, VPAD), jnp.float32),
        ),
    )(Lw0, W1, Lb0[:, None], Lw1, W0, Lb1[:, None])


def _gather_body(pt0_hbm, pt1_hbm, idx0_hbm, idx1_hbm, o0_hbm, o1_hbm,
                 idx_buf, pt_bufs, o_bufs, pt_sems, o_sems):
    arm = lax.axis_index("c")
    s = lax.axis_index("s")
    g = s // 2
    h = s % 2

    def run(pt_hbm, idx_hbm, o_hbm):
        pltpu.sync_copy(idx_hbm.at[pl.ds(h * HALF, HALF)], idx_buf)
        col0 = h * HALF

        def blk_of(kb):
            return jnp.minimum(g + NGRP * kb, NBLK - 1)

        def stage(kb, pb):
            return pltpu.make_async_copy(
                pt_hbm.at[pl.ds(blk_of(kb) * RB, RB)], pt_bufs[pb],
                pt_sems[pb])

        def write(kb, c, b):
            return pltpu.make_async_copy(
                o_bufs[b],
                o_hbm.at[pl.ds(blk_of(kb) * RB, RB),
                         pl.ds(col0 + c * CCH, CCH)],
                o_sems[b])

        def compute(c, pb, b):
            @plsc.parallel_loop(0, NIC, step=1, unroll=2)
            def _(ic):
                ivec = idx_buf[pl.ds(c * CCH + ic * L, L)]
                for jj in range(RB):
                    jvec = jnp.full((L,), jj, jnp.int32)
                    v = plsc.load_gather(pt_bufs[pb], [jvec, ivec])
                    o_bufs[b][jj, pl.ds(ic * L, L)] = v

        stage(0, 0).start()

        def pair(k, carry):
            for b in range(2):
                kb = k * 2 + b
                stage(kb, b).wait()

                @pl.when(kb + 1 < KMAX)
                def _():
                    stage(kb + 1, 1 - b).start()

                for c in range(NCH):
                    ob = c % 2
                    if c >= 2:
                        write(kb, c - 2, ob).wait()
                    else:
                        @pl.when(kb >= 1)
                        def _():
                            write(kb - 1, c + NCH - 2, ob).wait()

                    compute(c, b, ob)
                    write(kb, c, ob).start()
            return carry

        lax.fori_loop(0, KMAX // 2, pair, 0)
        write(KMAX - 1, NCH - 2, 0).wait()
        write(KMAX - 1, NCH - 1, 1).wait()

    @pl.when(arm == 0)
    def _():
        run(pt0_hbm, idx0_hbm, o0_hbm)

    @pl.when(arm == 1)
    def _():
        run(pt1_hbm, idx1_hbm, o1_hbm)


@functools.partial(
    pl.kernel,
    out_type=(
        jax.ShapeDtypeStruct((VOCAB, BATCH), jnp.float32),
        jax.ShapeDtypeStruct((VOCAB, BATCH), jnp.float32),
    ),
    mesh=plsc.VectorSubcoreMesh(core_axis_name="c", subcore_axis_name="s"),
    compiler_params=pltpu.CompilerParams(needs_layout_passes=False,
                                         disable_bounds_checks=True),
    scratch_types=(
        pltpu.VMEM((HALF,), jnp.int32),
        pltpu.VMEM((RB, VPAD), jnp.float32),
        pltpu.VMEM((RB, VPAD), jnp.float32),
        pltpu.VMEM((RB, CCH), jnp.float32),
        pltpu.VMEM((RB, CCH), jnp.float32),
        pltpu.SemaphoreType.DMA,
        pltpu.SemaphoreType.DMA,
        pltpu.SemaphoreType.DMA,
        pltpu.SemaphoreType.DMA,
    ),
)
def _gather_predictions(pt0, pt1, idx0, idx1, o0, o1, idx_buf, ptb0, ptb1,
                        ob0, ob1, ptsem0, ptsem1, osem0, osem1):
    _gather_body(pt0, pt1, idx0, idx1, o0, o1, idx_buf, (ptb0, ptb1),
                 (ob0, ob1), (ptsem0, ptsem1), (osem0, osem1))


def kernel(context_word, W0, W1, Lw0, Lb0, Lw1, Lb1):
    PT0, PT1 = _decode_tables(W0, W1, Lw0, Lb0, Lw1, Lb1)
    idx0 = context_word[1].astype(jnp.int32)
    idx1 = context_word[0].astype(jnp.int32)
    predT0, predT1 = _gather_predictions(PT0, PT1, idx0, idx1)
    return (W0, W1, predT0.T, predT1.T)


# speedup vs baseline: 1.0185x; 1.0185x over previous
import functools

import jax
import jax.numpy as jnp
from jax import lax
from jax.experimental import pallas as pl
from jax.experimental.pallas import tpu as pltpu
from jax.experimental.pallas import tpu_sc as plsc

VOCAB = 1000
VPAD = 1024
EMB = 128
BATCH = 16384

L = 16
RB = 8
NBLK = VOCAB // RB
NGRP = 8
KMAX = 16
HALF = BATCH // 2
CCH = 4096
NCH = HALF // CCH
NIC = CCH // L


def _decode_kernel(lw0_ref, w1_ref, lb0_ref, lw1_ref, w0_ref, lb1_ref,
                   pt0_ref, pt1_ref):
    pt0_ref[:, :VOCAB] = (
        jax.lax.dot_general(lw0_ref[...], w1_ref[...],
                            (((1,), (1,)), ((), ())),
                            preferred_element_type=jnp.float32)
        + lb0_ref[...]
    )
    pt1_ref[:, :VOCAB] = (
        jax.lax.dot_general(lw1_ref[...], w0_ref[...],
                            (((1,), (1,)), ((), ())),
                            preferred_element_type=jnp.float32)
        + lb1_ref[...]
    )


def _decode_tables(W0, W1, Lw0, Lb0, Lw1, Lb1):
    return pl.pallas_call(
        _decode_kernel,
        out_shape=(
            jax.ShapeDtypeStruct((VOCAB, VPAD), jnp.float32),
            jax.ShapeDtypeStruct((VOCAB, VPAD), jnp.float32),
        ),
    )(Lw0, W1, Lb0[:, None], Lw1, W0, Lb1[:, None])


def _gather_body(pt0_hbm, pt1_hbm, idx0_hbm, idx1_hbm, o0_hbm, o1_hbm,
                 idx_buf, pt_bufs, o_bufs, pt_sems, o_sems):
    arm = lax.axis_index("c")
    s = lax.axis_index("s")
    g = s // 2
    h = s % 2

    def run(pt_hbm, idx_hbm, o_hbm):
        pltpu.sync_copy(idx_hbm.at[pl.ds(h * HALF, HALF)], idx_buf)
        col0 = h * HALF

        def blk_of(kb):
            return jnp.minimum(g + NGRP * kb, NBLK - 1)

        def stage(kb, pb):
            return pltpu.make_async_copy(
                pt_hbm.at[pl.ds(blk_of(kb) * RB, RB)], pt_bufs[pb],
                pt_sems[pb])

        def write(kb, c, b):
            return pltpu.make_async_copy(
                o_bufs[b],
                o_hbm.at[pl.ds(blk_of(kb) * RB, RB),
                         pl.ds(col0 + c * CCH, CCH)],
                o_sems[b])

        def compute(c, pb, b):
            @plsc.parallel_loop(0, NIC, step=1, unroll=2)
            def _(ic):
                ivec = idx_buf[pl.ds(c * CCH + ic * L, L)]
                for jj in range(RB):
                    jvec = jnp.full((L,), jj, jnp.int32)
                    v = plsc.load_gather(pt_bufs[pb], [jvec, ivec])
                    o_bufs[b][jj, pl.ds(ic * L, L)] = v

        stage(0, 0).start()

        def pair(k, carry):
            for b in range(2):
                kb = k * 2 + b
                stage(kb, b).wait()

                @pl.when(kb + 1 < KMAX)
                def _():
                    stage(kb + 1, 1 - b).start()

                for c in range(NCH):
                    @pl.when(kb >= 1)
                    def _():
                        write(kb - 1, c, c).wait()

                    compute(c, b, c)
                    write(kb, c, c).start()
            return carry

        lax.fori_loop(0, KMAX // 2, pair, 0)
        write(KMAX - 1, 0, 0).wait()
        write(KMAX - 1, 1, 1).wait()

    @pl.when(arm == 0)
    def _():
        run(pt0_hbm, idx0_hbm, o0_hbm)

    @pl.when(arm == 1)
    def _():
        run(pt1_hbm, idx1_hbm, o1_hbm)


@functools.partial(
    pl.kernel,
    out_type=(
        jax.ShapeDtypeStruct((VOCAB, BATCH), jnp.float32),
        jax.ShapeDtypeStruct((VOCAB, BATCH), jnp.float32),
    ),
    mesh=plsc.VectorSubcoreMesh(core_axis_name="c", subcore_axis_name="s"),
    compiler_params=pltpu.CompilerParams(needs_layout_passes=False,
                                         disable_bounds_checks=True),
    scratch_types=(
        pltpu.VMEM((HALF,), jnp.int32),
        pltpu.VMEM((RB, VPAD), jnp.float32),
        pltpu.VMEM((RB, VPAD), jnp.float32),
        pltpu.VMEM((RB, CCH), jnp.float32),
        pltpu.VMEM((RB, CCH), jnp.float32),
        pltpu.SemaphoreType.DMA,
        pltpu.SemaphoreType.DMA,
        pltpu.SemaphoreType.DMA,
        pltpu.SemaphoreType.DMA,
    ),
)
def _gather_predictions(pt0, pt1, idx0, idx1, o0, o1, idx_buf, ptb0, ptb1,
                        ob0, ob1, ptsem0, ptsem1, osem0, osem1):
    _gather_body(pt0, pt1, idx0, idx1, o0, o1, idx_buf, (ptb0, ptb1),
                 (ob0, ob1), (ptsem0, ptsem1), (osem0, osem1))


def kernel(context_word, W0, W1, Lw0, Lb0, Lw1, Lb1):
    PT0, PT1 = _decode_tables(W0, W1, Lw0, Lb0, Lw1, Lb1)
    idx0 = context_word[1].astype(jnp.int32)
    idx1 = context_word[0].astype(jnp.int32)
    predT0, predT1 = _gather_predictions(PT0, PT1, idx0, idx1)
    return (W0, W1, predT0.T, predT1.T)
